# baseline (speedup 1.0000x reference)
import functools
import numpy as np
import jax
import jax.numpy as jnp
from jax import lax
from jax.experimental import pallas as pl
from jax.experimental.pallas import tpu as pltpu
from jax.experimental.pallas import tpu_sc as plsc

B = 1
S = 2048
D = 1024
H = 16
HD = D // H
HEAVY = int(0.1 * S)
RECENT = int(0.1 * S)
CACHE = HEAVY + RECENT
NCAND = HEAVY + 1
CV = 13
SPAD = 2064
MASK_MIN = -1e9
ROW_BLK = 256
NBLK = S // ROW_BLK
I32_MAX = np.int32(2**31 - 1)


def _proj_rope_body(h_ref, wt_ref, p_ref, cos_ref, sin_ref, o_ref):
    hb = h_ref[...]
    q = jnp.dot(hb, wt_ref[...], preferred_element_type=jnp.float32)
    qp = jnp.dot(q, p_ref[...], preferred_element_type=jnp.float32)
    o_ref[...] = q * cos_ref[...] + qp * sin_ref[...]


def _proj_body(h_ref, wt_ref, o_ref):
    o_ref[...] = jnp.dot(h_ref[...], wt_ref[...],
                         preferred_element_type=jnp.float32)


def _scores_body(q_ref, kt_ref, s_ref, ss_ref):
    i = pl.program_id(1)
    q = q_ref[0]
    kt = kt_ref[0]
    w = jnp.dot(q, kt, preferred_element_type=jnp.float32) * (1.0 / np.sqrt(HD))
    rows = lax.broadcasted_iota(jnp.int32, (ROW_BLK, S), 0) + i * ROW_BLK
    cols = lax.broadcasted_iota(jnp.int32, (ROW_BLK, S), 1)
    w = w + jnp.where(cols <= rows, 0.0, MASK_MIN)
    m = jnp.max(w, axis=-1, keepdims=True)
    e = jnp.exp(w - m)
    sc = e / jnp.sum(e, axis=-1, keepdims=True)
    s_ref[0] = sc
    init_rows = jnp.where(rows < CACHE, 1.0, 0.0)
    ss_ref[0, 0, 0] = jnp.sum(sc * init_rows, axis=0)


def _final_attn_body(q_ref, kt_ref, ev_ref, v_ref, o_ref):
    i = pl.program_id(1)
    w = jnp.dot(q_ref[0], kt_ref[0],
                preferred_element_type=jnp.float32) * (1.0 / np.sqrt(HD))
    rows = lax.broadcasted_iota(jnp.int32, (ROW_BLK, S), 0) + i * ROW_BLK
    cols = lax.broadcasted_iota(jnp.int32, (ROW_BLK, S), 1)
    w = w + jnp.where(cols <= rows, 0.0, MASK_MIN)
    m = jnp.max(w, axis=-1, keepdims=True)
    e = jnp.exp(w - m)
    ev = ev_ref[0]
    keep = rows <= ev
    mf = jnp.where(keep, e, 0.0)
    probs = mf / jnp.sum(mf, axis=-1, keepdims=True)
    o_ref[0] = jnp.dot(probs, v_ref[0], preferred_element_type=jnp.float32)


def _sc_evict_body(scores_hbm, ssinit_hbm, evict_hbm,
                   ss_v, row_a, row_b, row_c, row_d, cand_v, evict_v,
                   sem_a, sem_b, sem_c, sem_d):
    cid = lax.axis_index("c")
    sid = lax.axis_index("s")
    h = sid * 2 + cid
    lanes = lax.iota(jnp.int32, 16)
    lane0 = lanes == 0
    inf16 = jnp.full((16,), jnp.inf, jnp.float32)
    big16 = jnp.full((16,), I32_MAX, jnp.int32)

    @pl.when(h < H)
    def _():
        pltpu.sync_copy(ssinit_hbm.at[h], ss_v)
        row_a[pl.ds(2048, 16)] = jnp.zeros((16,), jnp.float32)
        row_b[pl.ds(2048, 16)] = jnp.zeros((16,), jnp.float32)
        row_c[pl.ds(2048, 16)] = jnp.zeros((16,), jnp.float32)
        row_d[pl.ds(2048, 16)] = jnp.zeros((16,), jnp.float32)
        for i in range(CV):
            vals = lanes + (16 * i)
            if i == CV - 1:
                vals = jnp.where(lanes < (NCAND - 16 * i), vals, S - 1)
            cand_v[pl.ds(16 * i, 16)] = vals
        for i in range(S // 16):
            evict_v[pl.ds(16 * i, 16)] = big16

        def fetch(t, row_v, sem):
            pltpu.make_async_copy(scores_hbm.at[h, t],
                                  row_v.at[pl.ds(0, S)], sem).start()

        def wait(t, row_v, sem):
            pltpu.make_async_copy(scores_hbm.at[h, t],
                                  row_v.at[pl.ds(0, S)], sem).wait()

        def process(t, row_v):
            base = t - (RECENT - 1)
            cpos = [cand_v[pl.ds(16 * i, 16)] for i in range(CV)]
            s_c = [plsc.load_gather(row_v, [cpos[i]]) for i in range(CV)]
            ridx = [jnp.full((16,), base + 16 * i, jnp.int32) + lanes
                    for i in range(CV)]
            s_r = [plsc.load_gather(row_v, [ridx[i]]) for i in range(CV)]
            acc = s_c[0]
            for i in range(1, CV):
                acc = acc + s_c[i]
            for i in range(CV):
                acc = acc + s_r[i]
            tot = jnp.sum(acc)
            inv_v = 1.0 / jnp.full((16,), tot, jnp.float32)
            gc = []
            for i in range(CV):
                g = plsc.load_gather(ss_v, [cpos[i]])
                g = g + s_c[i] * inv_v
                plsc.store_scatter(ss_v, [cpos[i]], g)
                gc.append(g)
            for i in range(CV):
                g = plsc.load_gather(ss_v, [ridx[i]])
                plsc.store_scatter(ss_v, [ridx[i]], g + s_r[i] * inv_v)
            gc[CV - 1] = jnp.where(lanes < (NCAND - 16 * (CV - 1)),
                                   gc[CV - 1], inf16)
            mn = gc[0]
            for i in range(1, CV):
                mn = jnp.minimum(mn, gc[i])
            minv_v = jnp.full((16,), jnp.min(mn), jnp.float32)
            cacc = big16
            for i in range(CV):
                combo = lax.shift_left(cpos[i], 8) + (lanes + 16 * i)
                cacc = jnp.minimum(cacc,
                                   jnp.where(gc[i] == minv_v, combo, big16))
            c_v = jnp.full((16,), jnp.min(cacc), jnp.int32)
            p_v = lax.shift_right_logical(c_v, 8)
            k_v = c_v & 255
            plsc.store_scatter(evict_v, [p_v],
                               jnp.full((16,), t, jnp.int32), mask=lane0)
            plsc.store_scatter(cand_v, [k_v],
                               jnp.full((16,), base, jnp.int32), mask=lane0)

        bufs = [(row_a, sem_a), (row_b, sem_b), (row_c, sem_c), (row_d, sem_d)]
        for off, (rv, sm) in enumerate(bufs):
            fetch(CACHE + off, rv, sm)

        def quad(i, carry):
            t = CACHE + 4 * i
            for off, (rv, sm) in enumerate(bufs):
                wait(t + off, rv, sm)
                process(t + off, rv)
                fetch(t + off + 4, rv, sm)
            return carry

        nquads = (S - 1 - CACHE) // 4
        lax.fori_loop(0, nquads, quad, 0)
        t_last = CACHE + 4 * nquads
        for off in range(S - 1 - CACHE - 4 * nquads):
            rv, sm = bufs[off]
            wait(t_last + off, rv, sm)
            process(t_last + off, rv)
        rv, sm = bufs[3]
        wait(t_last + 3, rv, sm)
        pltpu.sync_copy(evict_v, evict_hbm.at[h])


def _sc_evict(scores, ssinit):
    mesh = plsc.VectorSubcoreMesh(core_axis_name="c", subcore_axis_name="s")
    k = functools.partial(
        pl.kernel, mesh=mesh,
        compiler_params=pltpu.CompilerParams(needs_layout_passes=False),
        out_type=jax.ShapeDtypeStruct((H, S), jnp.int32),
        scratch_types=[
            pltpu.VMEM((SPAD,), jnp.float32),
            pltpu.VMEM((SPAD,), jnp.float32),
            pltpu.VMEM((SPAD,), jnp.float32),
            pltpu.VMEM((SPAD,), jnp.float32),
            pltpu.VMEM((SPAD,), jnp.float32),
            pltpu.VMEM((16 * CV,), jnp.int32),
            pltpu.VMEM((S,), jnp.int32),
            pltpu.SemaphoreType.DMA,
            pltpu.SemaphoreType.DMA,
            pltpu.SemaphoreType.DMA,
            pltpu.SemaphoreType.DMA,
        ],
    )(_sc_evict_body)
    return k(scores, ssinit)


def _rot_perm():
    p = np.zeros((D, D), np.float32)
    for h in range(H):
        o = h * HD
        half = HD // 2
        for r in range(half):
            p[o + half + r, o + r] = -1.0
            p[o + r, o + half + r] = 1.0
    return jnp.asarray(p)


def _cos_sin():
    inv_freq = 1.0 / (10000.0 ** (np.arange(0, HD, 2, dtype=np.float32) / HD))
    t = np.arange(S, dtype=np.float32)
    freqs = np.outer(t, inv_freq)
    emb = np.concatenate([freqs, freqs], axis=-1)
    cos = np.tile(np.cos(emb), (1, H)).astype(np.float32)
    sin = np.tile(np.sin(emb), (1, H)).astype(np.float32)
    return jnp.asarray(cos), jnp.asarray(sin)


def kernel(hidden_states, attention_mask, position_ids, Wq, Wk, Wv, Wo):
    hs = hidden_states[0]
    cosT, sinT = _cos_sin()
    P = _rot_perm()

    rope_call = pl.pallas_call(
        _proj_rope_body,
        grid=(NBLK,),
        in_specs=[
            pl.BlockSpec((ROW_BLK, D), lambda i: (i, 0)),
            pl.BlockSpec((D, D), lambda i: (0, 0)),
            pl.BlockSpec((D, D), lambda i: (0, 0)),
            pl.BlockSpec((ROW_BLK, D), lambda i: (i, 0)),
            pl.BlockSpec((ROW_BLK, D), lambda i: (i, 0)),
        ],
        out_specs=pl.BlockSpec((ROW_BLK, D), lambda i: (i, 0)),
        out_shape=jax.ShapeDtypeStruct((S, D), jnp.float32),
    )
    q = rope_call(hs, Wq.T, P, cosT, sinT)
    kk = rope_call(hs, Wk.T, P, cosT, sinT)

    v = pl.pallas_call(
        _proj_body,
        grid=(NBLK,),
        in_specs=[
            pl.BlockSpec((ROW_BLK, D), lambda i: (i, 0)),
            pl.BlockSpec((D, D), lambda i: (0, 0)),
        ],
        out_specs=pl.BlockSpec((ROW_BLK, D), lambda i: (i, 0)),
        out_shape=jax.ShapeDtypeStruct((S, D), jnp.float32),
    )(hs, Wv.T)

    qh = jnp.transpose(q.reshape(S, H, HD), (1, 0, 2))
    kht = jnp.transpose(kk.reshape(S, H, HD), (1, 2, 0))
    vh = jnp.transpose(v.reshape(S, H, HD), (1, 0, 2))

    scores, ss_parts = pl.pallas_call(
        _scores_body,
        grid=(H, NBLK),
        in_specs=[
            pl.BlockSpec((1, ROW_BLK, HD), lambda h, i: (h, i, 0)),
            pl.BlockSpec((1, HD, S), lambda h, i: (h, 0, 0)),
        ],
        out_specs=[
            pl.BlockSpec((1, ROW_BLK, S), lambda h, i: (h, i, 0)),
            pl.BlockSpec((1, 1, 1, S), lambda h, i: (h, i, 0, 0)),
        ],
        out_shape=[
            jax.ShapeDtypeStruct((H, S, S), jnp.float32),
            jax.ShapeDtypeStruct((H, NBLK, 1, S), jnp.float32),
        ],
    )(qh, kht)

    ss_init = jnp.sum(ss_parts[:, :, 0], axis=1)
    ss_init = jnp.pad(ss_init, ((0, 0), (0, SPAD - S)))

    evict = _sc_evict(scores, ss_init)
    evict3 = evict.reshape(H, 1, S)

    ctx = pl.pallas_call(
        _final_attn_body,
        grid=(H, NBLK),
        in_specs=[
            pl.BlockSpec((1, ROW_BLK, HD), lambda h, i: (h, i, 0)),
            pl.BlockSpec((1, HD, S), lambda h, i: (h, 0, 0)),
            pl.BlockSpec((1, 1, S), lambda h, i: (h, 0, 0)),
            pl.BlockSpec((1, S, HD), lambda h, i: (h, 0, 0)),
        ],
        out_specs=pl.BlockSpec((1, ROW_BLK, HD), lambda h, i: (h, i, 0)),
        out_shape=jax.ShapeDtypeStruct((H, S, HD), jnp.float32),
    )(qh, kht, evict3, vh)

    merged = jnp.transpose(ctx, (1, 0, 2)).reshape(S, D)

    out = pl.pallas_call(
        _proj_body,
        grid=(NBLK,),
        in_specs=[
            pl.BlockSpec((ROW_BLK, D), lambda i: (i, 0)),
            pl.BlockSpec((D, D), lambda i: (0, 0)),
        ],
        out_specs=pl.BlockSpec((ROW_BLK, D), lambda i: (i, 0)),
        out_shape=jax.ShapeDtypeStruct((S, D), jnp.float32),
    )(merged, Wo.T)

    return out[None]


# speedup vs baseline: 1.0756x; 1.0756x over previous
import functools
import numpy as np
import jax
import jax.numpy as jnp
from jax import lax
from jax.experimental import pallas as pl
from jax.experimental.pallas import tpu as pltpu
from jax.experimental.pallas import tpu_sc as plsc

B = 1
S = 2048
D = 1024
H = 16
HD = D // H
HEAVY = int(0.1 * S)
RECENT = int(0.1 * S)
CACHE = HEAVY + RECENT
NCAND = HEAVY + 1
CV = 13
SPAD = 2064
MASK_MIN = -1e9
ROW_BLK = 256
NBLK = S // ROW_BLK
I32_MAX = np.int32(2**31 - 1)


def _proj_rope_body(h_ref, wt_ref, p_ref, cos_ref, sin_ref, o_ref):
    hb = h_ref[...]
    q = jnp.dot(hb, wt_ref[...], preferred_element_type=jnp.float32)
    qp = jnp.dot(q, p_ref[...], preferred_element_type=jnp.float32)
    o_ref[...] = q * cos_ref[...] + qp * sin_ref[...]


def _proj_body(h_ref, wt_ref, o_ref):
    o_ref[...] = jnp.dot(h_ref[...], wt_ref[...],
                         preferred_element_type=jnp.float32)


def _scores_body(q_ref, kt_ref, s_ref, ss_ref):
    i = pl.program_id(1)
    q = q_ref[0]
    kt = kt_ref[0]
    w = jnp.dot(q, kt, preferred_element_type=jnp.float32) * (1.0 / np.sqrt(HD))
    rows = lax.broadcasted_iota(jnp.int32, (ROW_BLK, S), 0) + i * ROW_BLK
    cols = lax.broadcasted_iota(jnp.int32, (ROW_BLK, S), 1)
    w = w + jnp.where(cols <= rows, 0.0, MASK_MIN)
    m = jnp.max(w, axis=-1, keepdims=True)
    e = jnp.exp(w - m)
    sc = e / jnp.sum(e, axis=-1, keepdims=True)
    s_ref[0] = sc
    init_rows = jnp.where(rows < CACHE, 1.0, 0.0)
    ss_ref[0, 0, 0] = jnp.sum(sc * init_rows, axis=0)


def _final_attn_body(s_ref, ev_ref, v_ref, o_ref):
    i = pl.program_id(1)
    sc = s_ref[0]
    ev = ev_ref[0]
    rows = lax.broadcasted_iota(jnp.int32, (ROW_BLK, S), 0) + i * ROW_BLK
    keep = rows <= ev
    mf = jnp.where(keep, sc, 0.0)
    probs = mf / jnp.sum(mf, axis=-1, keepdims=True)
    o_ref[0] = jnp.dot(probs, v_ref[0], preferred_element_type=jnp.float32)


def _sc_evict_body(scores_hbm, ssinit_hbm, evict_hbm,
                   ss_v, row_a, row_b, row_c, row_d, cand_v, evict_v,
                   sem_a, sem_b, sem_c, sem_d):
    cid = lax.axis_index("c")
    sid = lax.axis_index("s")
    h = sid * 2 + cid
    lanes = lax.iota(jnp.int32, 16)
    lane0 = lanes == 0
    inf16 = jnp.full((16,), jnp.inf, jnp.float32)
    big16 = jnp.full((16,), I32_MAX, jnp.int32)

    @pl.when(h < H)
    def _():
        pltpu.sync_copy(ssinit_hbm.at[h], ss_v)
        row_a[pl.ds(2048, 16)] = jnp.zeros((16,), jnp.float32)
        row_b[pl.ds(2048, 16)] = jnp.zeros((16,), jnp.float32)
        row_c[pl.ds(2048, 16)] = jnp.zeros((16,), jnp.float32)
        row_d[pl.ds(2048, 16)] = jnp.zeros((16,), jnp.float32)
        for i in range(CV):
            vals = lanes + (16 * i)
            if i == CV - 1:
                vals = jnp.where(lanes < (NCAND - 16 * i), vals, S - 1)
            cand_v[pl.ds(16 * i, 16)] = vals
        for i in range(S // 16):
            evict_v[pl.ds(16 * i, 16)] = big16

        def fetch(t, row_v, sem):
            pltpu.make_async_copy(scores_hbm.at[h, t],
                                  row_v.at[pl.ds(0, S)], sem).start()

        def wait(t, row_v, sem):
            pltpu.make_async_copy(scores_hbm.at[h, t],
                                  row_v.at[pl.ds(0, S)], sem).wait()

        def process(t, row_v):
            base = t - (RECENT - 1)
            cpos = [cand_v[pl.ds(16 * i, 16)] for i in range(CV)]
            s_c = [plsc.load_gather(row_v, [cpos[i]]) for i in range(CV)]
            ridx = [jnp.full((16,), base + 16 * i, jnp.int32) + lanes
                    for i in range(CV)]
            s_r = [plsc.load_gather(row_v, [ridx[i]]) for i in range(CV)]
            acc = s_c[0]
            for i in range(1, CV):
                acc = acc + s_c[i]
            for i in range(CV):
                acc = acc + s_r[i]
            tot = jnp.sum(acc)
            inv_v = 1.0 / jnp.full((16,), tot, jnp.float32)
            gc = []
            for i in range(CV):
                g = plsc.load_gather(ss_v, [cpos[i]])
                g = g + s_c[i] * inv_v
                plsc.store_scatter(ss_v, [cpos[i]], g)
                gc.append(g)
            for i in range(CV):
                g = plsc.load_gather(ss_v, [ridx[i]])
                plsc.store_scatter(ss_v, [ridx[i]], g + s_r[i] * inv_v)
            gc[CV - 1] = jnp.where(lanes < (NCAND - 16 * (CV - 1)),
                                   gc[CV - 1], inf16)
            mn = gc[0]
            for i in range(1, CV):
                mn = jnp.minimum(mn, gc[i])
            minv_v = jnp.full((16,), jnp.min(mn), jnp.float32)
            cacc = big16
            for i in range(CV):
                combo = lax.shift_left(cpos[i], 8) + (lanes + 16 * i)
                cacc = jnp.minimum(cacc,
                                   jnp.where(gc[i] == minv_v, combo, big16))
            c_v = jnp.full((16,), jnp.min(cacc), jnp.int32)
            p_v = lax.shift_right_logical(c_v, 8)
            k_v = c_v & 255
            plsc.store_scatter(evict_v, [p_v],
                               jnp.full((16,), t, jnp.int32), mask=lane0)
            plsc.store_scatter(cand_v, [k_v],
                               jnp.full((16,), base, jnp.int32), mask=lane0)

        bufs = [(row_a, sem_a), (row_b, sem_b), (row_c, sem_c), (row_d, sem_d)]
        for off, (rv, sm) in enumerate(bufs):
            fetch(CACHE + off, rv, sm)

        def quad(i, carry):
            t = CACHE + 4 * i
            for off, (rv, sm) in enumerate(bufs):
                wait(t + off, rv, sm)
                process(t + off, rv)
                fetch(t + off + 4, rv, sm)
            return carry

        nquads = (S - 1 - CACHE) // 4
        lax.fori_loop(0, nquads, quad, 0)
        t_last = CACHE + 4 * nquads
        for off in range(S - 1 - CACHE - 4 * nquads):
            rv, sm = bufs[off]
            wait(t_last + off, rv, sm)
            process(t_last + off, rv)
        rv, sm = bufs[3]
        wait(t_last + 3, rv, sm)
        pltpu.sync_copy(evict_v, evict_hbm.at[h])


def _sc_evict(scores, ssinit):
    mesh = plsc.VectorSubcoreMesh(core_axis_name="c", subcore_axis_name="s")
    k = functools.partial(
        pl.kernel, mesh=mesh,
        compiler_params=pltpu.CompilerParams(needs_layout_passes=False),
        out_type=jax.ShapeDtypeStruct((H, S), jnp.int32),
        scratch_types=[
            pltpu.VMEM((SPAD,), jnp.float32),
            pltpu.VMEM((SPAD,), jnp.float32),
            pltpu.VMEM((SPAD,), jnp.float32),
            pltpu.VMEM((SPAD,), jnp.float32),
            pltpu.VMEM((SPAD,), jnp.float32),
            pltpu.VMEM((16 * CV,), jnp.int32),
            pltpu.VMEM((S,), jnp.int32),
            pltpu.SemaphoreType.DMA,
            pltpu.SemaphoreType.DMA,
            pltpu.SemaphoreType.DMA,
            pltpu.SemaphoreType.DMA,
        ],
    )(_sc_evict_body)
    return k(scores, ssinit)


def _rot_perm():
    p = np.zeros((D, D), np.float32)
    for h in range(H):
        o = h * HD
        half = HD // 2
        for r in range(half):
            p[o + half + r, o + r] = -1.0
            p[o + r, o + half + r] = 1.0
    return jnp.asarray(p)


def _cos_sin():
    inv_freq = 1.0 / (10000.0 ** (np.arange(0, HD, 2, dtype=np.float32) / HD))
    t = np.arange(S, dtype=np.float32)
    freqs = np.outer(t, inv_freq)
    emb = np.concatenate([freqs, freqs], axis=-1)
    cos = np.tile(np.cos(emb), (1, H)).astype(np.float32)
    sin = np.tile(np.sin(emb), (1, H)).astype(np.float32)
    return jnp.asarray(cos), jnp.asarray(sin)


def kernel(hidden_states, attention_mask, position_ids, Wq, Wk, Wv, Wo):
    hs = hidden_states[0]
    cosT, sinT = _cos_sin()
    P = _rot_perm()

    rope_call = pl.pallas_call(
        _proj_rope_body,
        grid=(NBLK,),
        in_specs=[
            pl.BlockSpec((ROW_BLK, D), lambda i: (i, 0)),
            pl.BlockSpec((D, D), lambda i: (0, 0)),
            pl.BlockSpec((D, D), lambda i: (0, 0)),
            pl.BlockSpec((ROW_BLK, D), lambda i: (i, 0)),
            pl.BlockSpec((ROW_BLK, D), lambda i: (i, 0)),
        ],
        out_specs=pl.BlockSpec((ROW_BLK, D), lambda i: (i, 0)),
        out_shape=jax.ShapeDtypeStruct((S, D), jnp.float32),
    )
    q = rope_call(hs, Wq.T, P, cosT, sinT)
    kk = rope_call(hs, Wk.T, P, cosT, sinT)

    v = pl.pallas_call(
        _proj_body,
        grid=(NBLK,),
        in_specs=[
            pl.BlockSpec((ROW_BLK, D), lambda i: (i, 0)),
            pl.BlockSpec((D, D), lambda i: (0, 0)),
        ],
        out_specs=pl.BlockSpec((ROW_BLK, D), lambda i: (i, 0)),
        out_shape=jax.ShapeDtypeStruct((S, D), jnp.float32),
    )(hs, Wv.T)

    qh = jnp.transpose(q.reshape(S, H, HD), (1, 0, 2))
    kht = jnp.transpose(kk.reshape(S, H, HD), (1, 2, 0))
    vh = jnp.transpose(v.reshape(S, H, HD), (1, 0, 2))

    scores, ss_parts = pl.pallas_call(
        _scores_body,
        grid=(H, NBLK),
        in_specs=[
            pl.BlockSpec((1, ROW_BLK, HD), lambda h, i: (h, i, 0)),
            pl.BlockSpec((1, HD, S), lambda h, i: (h, 0, 0)),
        ],
        out_specs=[
            pl.BlockSpec((1, ROW_BLK, S), lambda h, i: (h, i, 0)),
            pl.BlockSpec((1, 1, 1, S), lambda h, i: (h, i, 0, 0)),
        ],
        out_shape=[
            jax.ShapeDtypeStruct((H, S, S), jnp.float32),
            jax.ShapeDtypeStruct((H, NBLK, 1, S), jnp.float32),
        ],
    )(qh, kht)

    ss_init = jnp.sum(ss_parts[:, :, 0], axis=1)
    ss_init = jnp.pad(ss_init, ((0, 0), (0, SPAD - S)))

    evict = _sc_evict(scores, ss_init)
    evict3 = evict.reshape(H, 1, S)

    ctx = pl.pallas_call(
        _final_attn_body,
        grid=(H, NBLK),
        in_specs=[
            pl.BlockSpec((1, ROW_BLK, S), lambda h, i: (h, i, 0)),
            pl.BlockSpec((1, 1, S), lambda h, i: (h, 0, 0)),
            pl.BlockSpec((1, S, HD), lambda h, i: (h, 0, 0)),
        ],
        out_specs=pl.BlockSpec((1, ROW_BLK, HD), lambda h, i: (h, i, 0)),
        out_shape=jax.ShapeDtypeStruct((H, S, HD), jnp.float32),
    )(scores, evict3, vh)

    merged = jnp.transpose(ctx, (1, 0, 2)).reshape(S, D)

    out = pl.pallas_call(
        _proj_body,
        grid=(NBLK,),
        in_specs=[
            pl.BlockSpec((ROW_BLK, D), lambda i: (i, 0)),
            pl.BlockSpec((D, D), lambda i: (0, 0)),
        ],
        out_specs=pl.BlockSpec((ROW_BLK, D), lambda i: (i, 0)),
        out_shape=jax.ShapeDtypeStruct((S, D), jnp.float32),
    )(merged, Wo.T)

    return out[None]
